# baseline (speedup 1.0000x reference)
import math

import jax
import jax.numpy as jnp
import numpy as np
from jax import lax
from jax.experimental import pallas as pl
from jax.experimental.pallas import tpu as pltpu
from jax.experimental.pallas import tpu_sc as plsc

DIM = 192
NUM_HEADS = 8
HD = DIM // NUM_HEADS
WIN = 2
NUM_VALUES = 4
TEMP = 10000.0
H = 224
W = 224
HW = H * W
C = 96
NPF = 48
NFREQ = 24

NW = 32
PPW = HW // NW
CB = 16
NCHUNK = PPW // CB
NPAIR = NCHUNK // 2

KVP = 256
RB = 8
BA = RB * W
BC = RB * W


def _build_constants():
    perm = np.zeros(DIM, dtype=np.int32)
    for u in range(DIM // 16):
        for l in range(16):
            head = l if l < 8 else 15 - l
            d = 2 * u + (1 if l >= 8 else 0)
            perm[16 * u + l] = head * HD + d
    lanes = np.arange(16)
    pal = np.where(lanes < 8, lanes, 15 - lanes)
    head_of = pal[np.arange(DIM) % 16]

    scale2 = 2 * math.pi
    eps = 1e-06
    ones = np.ones((WIN, WIN), dtype=np.float64)
    y_emb = np.cumsum(ones, axis=0)
    x_emb = np.cumsum(ones, axis=1)
    y_emb = y_emb / (y_emb[-1:, :] + eps) * scale2
    x_emb = x_emb / (x_emb[:, -1:] + eps) * scale2
    dim_t = np.arange(NPF, dtype=np.float64)
    dim_t = TEMP ** (2 * (dim_t // 2) / NPF)
    pos_x = x_emb[..., None] / dim_t
    pos_y = y_emb[..., None] / dim_t
    pos_x = np.stack((np.sin(pos_x[..., 0::2]), np.cos(pos_x[..., 1::2])),
                     axis=3).reshape(WIN, WIN, NPF)
    pos_y = np.stack((np.sin(pos_y[..., 0::2]), np.cos(pos_y[..., 1::2])),
                     axis=3).reshape(WIN, WIN, NPF)
    pe_win = np.concatenate((pos_y, pos_x), axis=2).reshape(NUM_VALUES, 2 * NPF)

    pe_feat = np.zeros(2 * NPF, dtype=np.int32)
    for m in range(NFREQ):
        pe_feat[m] = 2 * m
        pe_feat[NFREQ + m] = 2 * m + 1
        pe_feat[2 * NFREQ + m] = NPF + 2 * m
        pe_feat[3 * NFREQ + m] = NPF + 2 * m + 1
    a_sel = np.zeros(2 * DIM // 2, dtype=np.int32)
    b_sel = np.zeros(2 * DIM // 2, dtype=np.int32)
    for q in range(2 * DIM // 32):
        for t in range(16):
            a_sel[16 * q + t] = 32 * q + t
            b_sel[16 * q + t] = 32 * q + 16 + t
    return perm, head_of, pal, pe_win.astype(np.float32), pe_feat, a_sel, b_sel


_PERM, _HEAD_OF, _PAL, _PE_WIN, _PE_FEAT, _ASEL, _BSEL = _build_constants()


def _phase_a_body(y4_ref, x4_ref, fl_ref, kvwa_ref, kvwb_ref, qw_ref,
                  qpew_ref, qb_ref, wkc_ref, kv_ref, q_ref, qkc_ref, idx_ref):
    i = pl.program_id(0)
    f32 = jnp.float32

    def rnbits(x):
        b = lax.bitcast_convert_type(x, jnp.int32)
        return b + 0x7FFF + (lax.shift_right_logical(b, 16) & 1)

    for r in range(RB):
        yr = y4_ref[0, :, r, :].astype(jnp.bfloat16)
        ra = lax.dot_general(
            yr, kvwa_ref[...], (((0,), (1,)), ((), ())),
            preferred_element_type=f32)
        rb = lax.dot_general(
            yr, kvwb_ref[...], (((0,), (1,)), ((), ())),
            preferred_element_type=f32)
        packed = (lax.shift_right_logical(rnbits(ra), 16)
                  | (rnbits(rb) & jnp.int32(-65536)))
        kv_ref[pl.ds(r * W, W), :] = jnp.concatenate(
            [packed, jnp.zeros((W, KVP - DIM), jnp.int32)], axis=1)

    p0 = i * BA
    lin = lax.broadcasted_iota(jnp.int32, (1, BA), 1) + p0
    r = lin // W
    cc = lin - r * W
    wx = cc.astype(f32) + fl_ref[0:1, :]
    wy = r.astype(f32) + fl_ref[1:2, :]
    fx = jnp.floor(wx)
    fy = jnp.floor(wy)
    ox = wx - fx
    oy = wy - fy
    ix = jnp.clip(fx, -1.0, W).astype(jnp.int32)
    iy = jnp.clip(fy, -1.0, H).astype(jnp.int32)
    rows = []
    for dy in range(WIN):
        for dx in range(WIN):
            rr = jnp.clip(iy + dy, 0, H - 1)
            cx = jnp.clip(ix + dx, 0, W - 1)
            rows.append(rr * W + cx)
    idx_ref[...] = jnp.concatenate(rows, axis=0)

    sc2 = 2 * math.pi
    a = oy * (sc2 / (WIN + 1e-06))
    b = ox * (sc2 / (WIN + 1e-06))
    di = lax.broadcasted_iota(jnp.int32, (NFREQ, 1), 0).astype(f32)
    invd = jnp.exp(di * (-2.0 * math.log(TEMP) / NPF))
    th_y = invd * a
    th_x = invd * b
    xpe = jnp.concatenate(
        [jnp.sin(th_y), jnp.cos(th_y), jnp.sin(th_x), jnp.cos(th_x)], axis=0)

    qpe = lax.dot_general(
        xpe.astype(jnp.bfloat16), qpew_ref[...], (((0,), (1,)), ((), ())),
        preferred_element_type=f32)
    for r in range(RB):
        q_ref[pl.ds(r * W, W), :] = (
            lax.dot_general(
                x4_ref[0, :, r, :].astype(jnp.bfloat16), qw_ref[...],
                (((0,), (1,)), ((), ())),
                preferred_element_type=f32)
            + qpe[r * W:(r + 1) * W, :] + qb_ref[...])
    qkc_ref[...] = lax.dot_general(
        q_ref[...], wkc_ref[...], (((1,), (0,)), ((), ())),
        preferred_element_type=f32)


def _phase_a(y4, x4, fl, kv_wa, kv_wb, qw, qpew, qb, wkc):
    grid = (HW // BA,)
    return pl.pallas_call(
        _phase_a_body,
        grid=grid,
        in_specs=[
            pl.BlockSpec((1, C, RB, W), lambda i: (0, 0, i, 0)),
            pl.BlockSpec((1, C, RB, W), lambda i: (0, 0, i, 0)),
            pl.BlockSpec((2, BA), lambda i: (0, i)),
            pl.BlockSpec((DIM, C), lambda i: (0, 0)),
            pl.BlockSpec((DIM, C), lambda i: (0, 0)),
            pl.BlockSpec((DIM, C), lambda i: (0, 0)),
            pl.BlockSpec((DIM, C), lambda i: (0, 0)),
            pl.BlockSpec((1, DIM), lambda i: (0, 0)),
            pl.BlockSpec((DIM, 4 * 16), lambda i: (0, 0)),
        ],
        out_specs=[
            pl.BlockSpec((BA, KVP), lambda i: (i, 0)),
            pl.BlockSpec((BA, DIM), lambda i: (i, 0)),
            pl.BlockSpec((BA, 4 * 16), lambda i: (i, 0)),
            pl.BlockSpec((4, BA), lambda i: (0, i)),
        ],
        out_shape=[
            jax.ShapeDtypeStruct((HW, KVP), jnp.int32),
            jax.ShapeDtypeStruct((HW, DIM), jnp.float32),
            jax.ShapeDtypeStruct((HW, 4 * 16), jnp.float32),
            jax.ShapeDtypeStruct((4, HW), jnp.int32),
        ],
    )(y4, x4, fl, kv_wa, kv_wb, qw, qpew, qb, wkc)


def _phase_b_body(kv_hbm, q_hbm, qkc_hbm, idx_hbm, out_hbm, att_hbm,
                  ix0, ix1, ix2, ix3,
                  r00, r01, r02, r03, r10, r11, r12, r13,
                  q0, q1, qk0, qk1, o0, o1, a0, a1,
                  sg0, sg1, ss0, ss1):
    wid = lax.axis_index("s") * 2 + lax.axis_index("c")
    base_w = wid * PPW
    idxb = (ix0, ix1, ix2, ix3)
    rows = ((r00, r01, r02, r03), (r10, r11, r12, r13))
    qb = (q0, q1)
    qkb = (qk0, qk1)
    ob = (o0, o1)
    ab = (a0, a1)
    gsem = (sg0, sg1)
    ssem = (ss0, ss1)

    for j in range(NUM_VALUES):
        pltpu.sync_copy(idx_hbm.at[pl.ds(j * HW + base_w, PPW)], idxb[j])

    def issue(k, s):
        kk = jnp.minimum(k, NCHUNK - 1)
        base = base_w + kk * CB
        for j in range(NUM_VALUES):
            iv = idxb[j][pl.ds(kk * CB, CB)]
            pltpu.async_copy(kv_hbm.at[iv], rows[s][j], gsem[s])
        pltpu.async_copy(q_hbm.at[pl.ds(base, CB)], qb[s], gsem[s])
        pltpu.async_copy(qkc_hbm.at[pl.ds(base, CB)], qkb[s], gsem[s])

    def wait_gathers(s):
        iv0 = idxb[0][pl.ds(0, CB)]
        for j in range(NUM_VALUES):
            pltpu.make_async_copy(kv_hbm.at[iv0], rows[s][j], gsem[s]).wait()
        pltpu.make_async_copy(q_hbm.at[pl.ds(base_w, CB)], qb[s],
                              gsem[s]).wait()
        pltpu.make_async_copy(qkc_hbm.at[pl.ds(base_w, CB)], qkb[s],
                              gsem[s]).wait()

    def wait_stores(s):
        pltpu.make_async_copy(ob[s], out_hbm.at[pl.ds(base_w, CB)],
                              ssem[s]).wait()
        pltpu.make_async_copy(ab[s], att_hbm.at[pl.ds(base_w, CB)],
                              ssem[s]).wait()

    def compute(base, s):
        rj = rows[s]
        q_v = qb[s]
        qkc_v = qkb[s]
        out_v = ob[s]
        att_v = ab[s]

        @plsc.parallel_loop(0, CB)
        def pix_body(p):
            qv = [q_v[p, pl.ds(16 * v, 16)] for v in range(12)]
            es = []
            mx = None
            for j in range(NUM_VALUES):
                acc = qkc_v[p, pl.ds(16 * j, 16)]
                for u in range(6):
                    kw = rj[j][p, pl.ds(16 * u, 16)]
                    ka = lax.bitcast_convert_type(
                        lax.shift_left(kw, 16), jnp.float32)
                    kb = lax.bitcast_convert_type(
                        kw & jnp.int32(-65536), jnp.float32)
                    acc = acc + qv[2 * u] * ka + qv[2 * u + 1] * kb
                lg = acc + lax.rev(acc, (0,))
                es.append(lg)
                mx = lg if mx is None else jnp.maximum(mx, lg)
            ssum = None
            for j in range(NUM_VALUES):
                e = jnp.exp(es[j] - mx)
                es[j] = e
                ssum = e if ssum is None else ssum + e
            rinv = 1.0 / ssum
            attn = []
            for j in range(NUM_VALUES):
                aj = es[j] * rinv
                attn.append(aj)
                att_v[p, pl.ds(16 * j, 16)] = aj
            for u in range(6):
                oa = None
                ob = None
                for j in range(NUM_VALUES):
                    vw = rj[j][p, pl.ds(96 + 16 * u, 16)]
                    va = lax.bitcast_convert_type(
                        lax.shift_left(vw, 16), jnp.float32)
                    vb = lax.bitcast_convert_type(
                        vw & jnp.int32(-65536), jnp.float32)
                    if oa is None:
                        oa = attn[j] * va
                        ob = attn[j] * vb
                    else:
                        oa = oa + attn[j] * va
                        ob = ob + attn[j] * vb
                out_v[p, pl.ds(32 * u, 16)] = oa
                out_v[p, pl.ds(32 * u + 16, 16)] = ob
        pltpu.async_copy(out_v, out_hbm.at[pl.ds(base, CB)], ssem[s])
        pltpu.async_copy(att_v, att_hbm.at[pl.ds(base, CB)], ssem[s])

    issue(0, 0)

    def pair_body(h, carry):
        base0 = base_w + (2 * h) * CB
        base1 = base0 + CB
        issue(2 * h + 1, 1)
        wait_gathers(0)

        @pl.when(h > 0)
        def _():
            wait_stores(0)

        compute(base0, 0)
        issue(2 * h + 2, 0)
        wait_gathers(1)

        @pl.when(h > 0)
        def _():
            wait_stores(1)

        compute(base1, 1)
        return carry

    lax.fori_loop(0, NPAIR, pair_body, 0)
    wait_gathers(0)
    wait_stores(0)
    wait_stores(1)


def _phase_b(kv, q, qkc, idx4):
    mesh = plsc.VectorSubcoreMesh(core_axis_name="c", subcore_axis_name="s")
    f = pl.kernel(
        _phase_b_body,
        out_type=[
            jax.ShapeDtypeStruct((HW, DIM), jnp.float32),
            jax.ShapeDtypeStruct((HW, 4 * 16), jnp.float32),
        ],
        mesh=mesh,
        scratch_types=(
            [pltpu.VMEM((PPW,), jnp.int32)] * 4
            + [pltpu.VMEM((CB, KVP), jnp.int32)] * 8
            + [pltpu.VMEM((CB, DIM), jnp.float32),
               pltpu.VMEM((CB, DIM), jnp.float32),
               pltpu.VMEM((CB, 4 * 16), jnp.float32),
               pltpu.VMEM((CB, 4 * 16), jnp.float32),
               pltpu.VMEM((CB, DIM), jnp.float32),
               pltpu.VMEM((CB, DIM), jnp.float32),
               pltpu.VMEM((CB, 4 * 16), jnp.float32),
               pltpu.VMEM((CB, 4 * 16), jnp.float32)]
            + [pltpu.SemaphoreType.DMA] * 4
        ),
    )
    return f(kv, q, qkc, idx4)


def _phase_c_body(o_ref, att_ref, wvc_ref, pt_ref, out_ref):
    t = o_ref[...] + lax.dot_general(
        att_ref[...], wvc_ref[...], (((1,), (0,)), ((), ())),
        preferred_element_type=jnp.float32)
    for r in range(RB):
        out_ref[0, :, r, :] = lax.dot_general(
            pt_ref[...], t[r * W:(r + 1) * W, :], (((1,), (1,)), ((), ())),
            preferred_element_type=jnp.float32)


def _phase_c(o, att, wvc, pt):
    grid = (HW // BC,)
    return pl.pallas_call(
        _phase_c_body,
        grid=grid,
        in_specs=[
            pl.BlockSpec((BC, DIM), lambda i: (i, 0)),
            pl.BlockSpec((BC, 4 * 16), lambda i: (i, 0)),
            pl.BlockSpec((4 * 16, DIM), lambda i: (0, 0)),
            pl.BlockSpec((DIM, DIM), lambda i: (0, 0)),
        ],
        out_specs=pl.BlockSpec((1, DIM, RB, W), lambda i: (0, 0, i, 0)),
        out_shape=jax.ShapeDtypeStruct((1, DIM, H, W), jnp.float32),
    )(o, att, wvc, pt)


@jax.jit
def kernel(y, x, flow, q_w, q_b, k_w, k_b, v_w, v_b):
    scale = HD ** (-0.5)
    perm = jnp.asarray(_PERM)
    head_of = _HEAD_OF
    pe_win = jnp.asarray(_PE_WIN)

    k_wp = k_w[perm, :]
    v_wp = v_w[perm, :]
    q_wp = q_w[perm, :] * scale
    q_bp = (q_b[perm] * scale).reshape(1, DIM)
    kc = pe_win @ k_wp.T + k_b[perm]
    vc = pe_win @ v_wp.T + v_b[perm]

    kv_w = jnp.concatenate([k_wp, v_wp], axis=0)
    kv_wa = kv_w[jnp.asarray(_ASEL), :].astype(jnp.bfloat16)
    kv_wb = kv_w[jnp.asarray(_BSEL), :].astype(jnp.bfloat16)
    q_w_pe = q_wp[:, jnp.asarray(_PE_FEAT)].astype(jnp.bfloat16)
    q_wp = q_wp.astype(jnp.bfloat16)

    lanes = np.arange(16)
    sel_kc = jnp.asarray(_HEAD_OF[:, None, None] == _PAL[None, None, :])
    wkc = jnp.where(sel_kc, 0.5 * kc.T[:, :, None], 0.0)
    wkc = wkc.reshape(DIM, NUM_VALUES * 16)

    sel_vc = jnp.asarray(
        (_PAL[None, :, None] == _HEAD_OF[None, None, :])
        & (lanes[None, :, None] < 8))
    wvc = jnp.where(sel_vc, vc[:, None, :], 0.0)
    wvc = wvc.reshape(NUM_VALUES * 16, DIM)

    pt = jnp.asarray(np.eye(DIM, dtype=np.float32)[_PERM].T)

    fl = flow.reshape(HW, 2).T

    kv, q, qkc, idx4 = _phase_a(y, x, fl, kv_wa, kv_wb, q_wp, q_w_pe,
                                q_bp, wkc)
    o, att = _phase_b(kv, q, qkc, idx4.reshape(NUM_VALUES * HW))
    return _phase_c(o, att, wvc, pt)


# speedup vs baseline: 1.1049x; 1.1049x over previous
import math

import jax
import jax.numpy as jnp
import numpy as np
from jax import lax
from jax.experimental import pallas as pl
from jax.experimental.pallas import tpu as pltpu
from jax.experimental.pallas import tpu_sc as plsc

DIM = 192
NUM_HEADS = 8
HD = DIM // NUM_HEADS
WIN = 2
NUM_VALUES = 4
TEMP = 10000.0
H = 224
W = 224
HW = H * W
C = 96
NPF = 48
NFREQ = 24

NW = 32
PPW = HW // NW
CB = 16
NCHUNK = PPW // CB
NPAIR = NCHUNK // 2

KVP = 384
RB = 8
BA = RB * W
BC = RB * W
NB = BA + 2 * RB
NBLK = HW // BA


def _build_constants():
    perm = np.zeros(DIM, dtype=np.int32)
    for u in range(DIM // 16):
        for l in range(16):
            head = l if l < 8 else 15 - l
            d = 2 * u + (1 if l >= 8 else 0)
            perm[16 * u + l] = head * HD + d
    lanes = np.arange(16)
    pal = np.where(lanes < 8, lanes, 15 - lanes)
    head_of = pal[np.arange(DIM) % 16]

    scale2 = 2 * math.pi
    eps = 1e-06
    ones = np.ones((WIN, WIN), dtype=np.float64)
    y_emb = np.cumsum(ones, axis=0)
    x_emb = np.cumsum(ones, axis=1)
    y_emb = y_emb / (y_emb[-1:, :] + eps) * scale2
    x_emb = x_emb / (x_emb[:, -1:] + eps) * scale2
    dim_t = np.arange(NPF, dtype=np.float64)
    dim_t = TEMP ** (2 * (dim_t // 2) / NPF)
    pos_x = x_emb[..., None] / dim_t
    pos_y = y_emb[..., None] / dim_t
    pos_x = np.stack((np.sin(pos_x[..., 0::2]), np.cos(pos_x[..., 1::2])),
                     axis=3).reshape(WIN, WIN, NPF)
    pos_y = np.stack((np.sin(pos_y[..., 0::2]), np.cos(pos_y[..., 1::2])),
                     axis=3).reshape(WIN, WIN, NPF)
    pe_win = np.concatenate((pos_y, pos_x), axis=2).reshape(NUM_VALUES, 2 * NPF)

    pe_feat = np.zeros(2 * NPF, dtype=np.int32)
    for m in range(NFREQ):
        pe_feat[m] = 2 * m
        pe_feat[NFREQ + m] = 2 * m + 1
        pe_feat[2 * NFREQ + m] = NPF + 2 * m
        pe_feat[3 * NFREQ + m] = NPF + 2 * m + 1
    a_sel = np.zeros(2 * DIM // 2, dtype=np.int32)
    b_sel = np.zeros(2 * DIM // 2, dtype=np.int32)
    for q in range(2 * DIM // 32):
        for t in range(16):
            a_sel[16 * q + t] = 32 * q + t
            b_sel[16 * q + t] = 32 * q + 16 + t
    return perm, head_of, pal, pe_win.astype(np.float32), pe_feat, a_sel, b_sel


_PERM, _HEAD_OF, _PAL, _PE_WIN, _PE_FEAT, _ASEL, _BSEL = _build_constants()


def _phase_a_body(y4_ref, x4_ref, fl_ref, kvwa_ref, kvwb_ref, qw_ref,
                  qpew_ref, qb_ref, wkc_ref, kv_ref, q_ref, qkc_ref, idx_ref):
    i = pl.program_id(0)
    f32 = jnp.float32

    def rnbits(x):
        b = lax.bitcast_convert_type(x, jnp.int32)
        return b + 0x7FFF + (lax.shift_right_logical(b, 16) & 1)

    specials = []
    for r in range(RB):
        ra = lax.dot_general(
            y4_ref[0, :, r, :], kvwa_ref[...], (((0,), (1,)), ((), ())),
            preferred_element_type=f32)
        rb = lax.dot_general(
            y4_ref[0, :, r, :], kvwb_ref[...], (((0,), (1,)), ((), ())),
            preferred_element_type=f32)
        packed = (lax.shift_right_logical(rnbits(ra), 16)
                  | (rnbits(rb) & jnp.int32(-65536)))
        shifted = jnp.concatenate(
            [packed[1:W, :], packed[W - 1:W, :]], axis=0)
        kv_ref[pl.ds(r * W, W), :] = jnp.concatenate(
            [packed, shifted], axis=1)
        sl = packed[0:1, :]
        sr = packed[W - 1:W, :]
        specials.append(jnp.concatenate([sl, sl], axis=1))
        specials.append(jnp.concatenate([sr, sr], axis=1))
    kv_ref[pl.ds(BA, 2 * RB), :] = jnp.concatenate(specials, axis=0)

    p0 = i * BA
    lin = lax.broadcasted_iota(jnp.int32, (1, BA), 1) + p0
    r = lin // W
    cc = lin - r * W
    wx = cc.astype(f32) + fl_ref[0:1, :]
    wy = r.astype(f32) + fl_ref[1:2, :]
    fx = jnp.floor(wx)
    fy = jnp.floor(wy)
    ox = wx - fx
    oy = wy - fy
    ix = jnp.clip(fx, -1.0, W).astype(jnp.int32)
    iy = jnp.clip(fy, -1.0, H).astype(jnp.int32)
    cs = jnp.clip(ix, 0, W - 2)
    clipped = (ix < 0) | (ix >= W - 1)
    side = (ix >= W - 1).astype(jnp.int32)
    rows = []
    for dy in range(WIN):
        rr = jnp.clip(iy + dy, 0, H - 1)
        blk = rr // RB
        off = rr - blk * RB
        main = blk * NB + off * W + cs
        spec = blk * NB + BA + 2 * off + side
        rows.append(jnp.where(clipped, spec, main))
    idx_ref[...] = jnp.concatenate(rows, axis=0)

    sc2 = 2 * math.pi
    a = oy * (sc2 / (WIN + 1e-06))
    b = ox * (sc2 / (WIN + 1e-06))
    di = lax.broadcasted_iota(jnp.int32, (NFREQ, 1), 0).astype(f32)
    invd = jnp.exp(di * (-2.0 * math.log(TEMP) / NPF))
    th_y = invd * a
    th_x = invd * b
    xpe = jnp.concatenate(
        [jnp.sin(th_y), jnp.cos(th_y), jnp.sin(th_x), jnp.cos(th_x)], axis=0)

    qpe = lax.dot_general(
        xpe, qpew_ref[...], (((0,), (1,)), ((), ())),
        preferred_element_type=f32)
    for r in range(RB):
        q_ref[pl.ds(r * W, W), :] = (
            lax.dot_general(
                x4_ref[0, :, r, :], qw_ref[...],
                (((0,), (1,)), ((), ())),
                preferred_element_type=f32)
            + qpe[r * W:(r + 1) * W, :] + qb_ref[...])
    qkc_ref[...] = lax.dot_general(
        q_ref[...], wkc_ref[...], (((1,), (0,)), ((), ())),
        preferred_element_type=f32)


def _phase_a(y4, x4, fl, kv_wa, kv_wb, qw, qpew, qb, wkc):
    grid = (HW // BA,)
    return pl.pallas_call(
        _phase_a_body,
        grid=grid,
        in_specs=[
            pl.BlockSpec((1, C, RB, W), lambda i: (0, 0, i, 0)),
            pl.BlockSpec((1, C, RB, W), lambda i: (0, 0, i, 0)),
            pl.BlockSpec((2, BA), lambda i: (0, i)),
            pl.BlockSpec((DIM, C), lambda i: (0, 0)),
            pl.BlockSpec((DIM, C), lambda i: (0, 0)),
            pl.BlockSpec((DIM, C), lambda i: (0, 0)),
            pl.BlockSpec((DIM, C), lambda i: (0, 0)),
            pl.BlockSpec((1, DIM), lambda i: (0, 0)),
            pl.BlockSpec((DIM, 4 * 16), lambda i: (0, 0)),
        ],
        out_specs=[
            pl.BlockSpec((NB, KVP), lambda i: (i, 0)),
            pl.BlockSpec((BA, DIM), lambda i: (i, 0)),
            pl.BlockSpec((BA, 4 * 16), lambda i: (i, 0)),
            pl.BlockSpec((2, BA), lambda i: (0, i)),
        ],
        out_shape=[
            jax.ShapeDtypeStruct((NBLK * NB, KVP), jnp.int32),
            jax.ShapeDtypeStruct((HW, DIM), jnp.float32),
            jax.ShapeDtypeStruct((HW, 4 * 16), jnp.float32),
            jax.ShapeDtypeStruct((2, HW), jnp.int32),
        ],
    )(y4, x4, fl, kv_wa, kv_wb, qw, qpew, qb, wkc)


def _phase_b_body(kv_hbm, q_hbm, qkc_hbm, idx_hbm, out_hbm, att_hbm,
                  ix0, ix1,
                  r00, r01, r10, r11,
                  q0, q1, qk0, qk1, o0, o1, a0, a1,
                  sg0, sg1, ss0, ss1):
    wid = lax.axis_index("s") * 2 + lax.axis_index("c")
    base_w = wid * PPW
    idxb = (ix0, ix1)
    rows = ((r00, r01), (r10, r11))
    qb = (q0, q1)
    qkb = (qk0, qk1)
    ob = (o0, o1)
    ab = (a0, a1)
    gsem = (sg0, sg1)
    ssem = (ss0, ss1)

    for j in range(WIN):
        pltpu.sync_copy(idx_hbm.at[pl.ds(j * HW + base_w, PPW)], idxb[j])

    def issue(k, s):
        kk = jnp.minimum(k, NCHUNK - 1)
        base = base_w + kk * CB
        for j in range(WIN):
            iv = idxb[j][pl.ds(kk * CB, CB)]
            pltpu.async_copy(kv_hbm.at[iv], rows[s][j], gsem[s])
        pltpu.async_copy(q_hbm.at[pl.ds(base, CB)], qb[s], gsem[s])
        pltpu.async_copy(qkc_hbm.at[pl.ds(base, CB)], qkb[s], gsem[s])

    def wait_gathers(s):
        iv0 = idxb[0][pl.ds(0, CB)]
        for j in range(WIN):
            pltpu.make_async_copy(kv_hbm.at[iv0], rows[s][j], gsem[s]).wait()
        pltpu.make_async_copy(q_hbm.at[pl.ds(base_w, CB)], qb[s],
                              gsem[s]).wait()
        pltpu.make_async_copy(qkc_hbm.at[pl.ds(base_w, CB)], qkb[s],
                              gsem[s]).wait()

    def wait_stores(s):
        pltpu.make_async_copy(ob[s], out_hbm.at[pl.ds(base_w, CB)],
                              ssem[s]).wait()
        pltpu.make_async_copy(ab[s], att_hbm.at[pl.ds(base_w, CB)],
                              ssem[s]).wait()

    def compute(base, s):
        rj = rows[s]
        q_v = qb[s]
        qkc_v = qkb[s]
        out_v = ob[s]
        att_v = ab[s]

        @plsc.parallel_loop(0, CB)
        def pix_body(p):
            qv = [q_v[p, pl.ds(16 * v, 16)] for v in range(12)]
            es = []
            mx = None
            for j in range(NUM_VALUES):
                acc = qkc_v[p, pl.ds(16 * j, 16)]
                half = DIM * (j & 1)
                for u in range(6):
                    kw = rj[j // 2][p, pl.ds(half + 16 * u, 16)]
                    ka = lax.bitcast_convert_type(
                        lax.shift_left(kw, 16), jnp.float32)
                    kb = lax.bitcast_convert_type(
                        kw & jnp.int32(-65536), jnp.float32)
                    acc = acc + qv[2 * u] * ka + qv[2 * u + 1] * kb
                lg = acc + lax.rev(acc, (0,))
                es.append(lg)
                mx = lg if mx is None else jnp.maximum(mx, lg)
            ssum = None
            for j in range(NUM_VALUES):
                e = jnp.exp(es[j] - mx)
                es[j] = e
                ssum = e if ssum is None else ssum + e
            rinv = 1.0 / ssum
            attn = []
            for j in range(NUM_VALUES):
                aj = es[j] * rinv
                attn.append(aj)
                att_v[p, pl.ds(16 * j, 16)] = aj
            for u in range(6):
                oa = None
                ob = None
                for j in range(NUM_VALUES):
                    vw = rj[j // 2][p, pl.ds(DIM * (j & 1) + 96 + 16 * u, 16)]
                    va = lax.bitcast_convert_type(
                        lax.shift_left(vw, 16), jnp.float32)
                    vb = lax.bitcast_convert_type(
                        vw & jnp.int32(-65536), jnp.float32)
                    if oa is None:
                        oa = attn[j] * va
                        ob = attn[j] * vb
                    else:
                        oa = oa + attn[j] * va
                        ob = ob + attn[j] * vb
                out_v[p, pl.ds(32 * u, 16)] = oa
                out_v[p, pl.ds(32 * u + 16, 16)] = ob
        pltpu.async_copy(out_v, out_hbm.at[pl.ds(base, CB)], ssem[s])
        pltpu.async_copy(att_v, att_hbm.at[pl.ds(base, CB)], ssem[s])

    issue(0, 0)

    def pair_body(h, carry):
        base0 = base_w + (2 * h) * CB
        base1 = base0 + CB
        issue(2 * h + 1, 1)
        wait_gathers(0)

        @pl.when(h > 0)
        def _():
            wait_stores(0)

        compute(base0, 0)
        issue(2 * h + 2, 0)
        wait_gathers(1)

        @pl.when(h > 0)
        def _():
            wait_stores(1)

        compute(base1, 1)
        return carry

    lax.fori_loop(0, NPAIR, pair_body, 0)
    wait_gathers(0)
    wait_stores(0)
    wait_stores(1)


def _phase_b(kv, q, qkc, idx4):
    mesh = plsc.VectorSubcoreMesh(core_axis_name="c", subcore_axis_name="s")
    f = pl.kernel(
        _phase_b_body,
        out_type=[
            jax.ShapeDtypeStruct((HW, DIM), jnp.float32),
            jax.ShapeDtypeStruct((HW, 4 * 16), jnp.float32),
        ],
        mesh=mesh,
        scratch_types=(
            [pltpu.VMEM((PPW,), jnp.int32)] * 2
            + [pltpu.VMEM((CB, KVP), jnp.int32)] * 4
            + [pltpu.VMEM((CB, DIM), jnp.float32),
               pltpu.VMEM((CB, DIM), jnp.float32),
               pltpu.VMEM((CB, 4 * 16), jnp.float32),
               pltpu.VMEM((CB, 4 * 16), jnp.float32),
               pltpu.VMEM((CB, DIM), jnp.float32),
               pltpu.VMEM((CB, DIM), jnp.float32),
               pltpu.VMEM((CB, 4 * 16), jnp.float32),
               pltpu.VMEM((CB, 4 * 16), jnp.float32)]
            + [pltpu.SemaphoreType.DMA] * 4
        ),
    )
    return f(kv, q, qkc, idx4)


def _phase_c_body(o_ref, att_ref, wvc_ref, pt_ref, out_ref):
    t = o_ref[...] + lax.dot_general(
        att_ref[...], wvc_ref[...], (((1,), (0,)), ((), ())),
        preferred_element_type=jnp.float32)
    for r in range(RB):
        out_ref[0, :, r, :] = lax.dot_general(
            pt_ref[...], t[r * W:(r + 1) * W, :], (((1,), (1,)), ((), ())),
            preferred_element_type=jnp.float32)


def _phase_c(o, att, wvc, pt):
    grid = (HW // BC,)
    return pl.pallas_call(
        _phase_c_body,
        grid=grid,
        in_specs=[
            pl.BlockSpec((BC, DIM), lambda i: (i, 0)),
            pl.BlockSpec((BC, 4 * 16), lambda i: (i, 0)),
            pl.BlockSpec((4 * 16, DIM), lambda i: (0, 0)),
            pl.BlockSpec((DIM, DIM), lambda i: (0, 0)),
        ],
        out_specs=pl.BlockSpec((1, DIM, RB, W), lambda i: (0, 0, i, 0)),
        out_shape=jax.ShapeDtypeStruct((1, DIM, H, W), jnp.float32),
    )(o, att, wvc, pt)


@jax.jit
def kernel(y, x, flow, q_w, q_b, k_w, k_b, v_w, v_b):
    scale = HD ** (-0.5)
    perm = jnp.asarray(_PERM)
    head_of = _HEAD_OF
    pe_win = jnp.asarray(_PE_WIN)

    k_wp = k_w[perm, :]
    v_wp = v_w[perm, :]
    q_wp = q_w[perm, :] * scale
    q_bp = (q_b[perm] * scale).reshape(1, DIM)
    kc = pe_win @ k_wp.T + k_b[perm]
    vc = pe_win @ v_wp.T + v_b[perm]

    kv_w = jnp.concatenate([k_wp, v_wp], axis=0)
    kv_wa = kv_w[jnp.asarray(_ASEL), :]
    kv_wb = kv_w[jnp.asarray(_BSEL), :]
    q_w_pe = q_wp[:, jnp.asarray(_PE_FEAT)]

    lanes = np.arange(16)
    sel_kc = jnp.asarray(_HEAD_OF[:, None, None] == _PAL[None, None, :])
    wkc = jnp.where(sel_kc, 0.5 * kc.T[:, :, None], 0.0)
    wkc = wkc.reshape(DIM, NUM_VALUES * 16)

    sel_vc = jnp.asarray(
        (_PAL[None, :, None] == _HEAD_OF[None, None, :])
        & (lanes[None, :, None] < 8))
    wvc = jnp.where(sel_vc, vc[:, None, :], 0.0)
    wvc = wvc.reshape(NUM_VALUES * 16, DIM)

    pt = jnp.asarray(np.eye(DIM, dtype=np.float32)[_PERM].T)

    fl = flow.reshape(HW, 2).T

    kv, q, qkc, idx4 = _phase_a(y, x, fl, kv_wa, kv_wb, q_wp, q_w_pe,
                                q_bp, wkc)
    o, att = _phase_b(kv, q, qkc, idx4.reshape(WIN * HW))
    return _phase_c(o, att, wvc, pt)
